# baseline (speedup 1.0000x reference)
import functools

import jax
import jax.numpy as jnp
from jax import lax
from jax.experimental import pallas as pl
from jax.experimental.pallas import tpu as pltpu
from jax.experimental.pallas import tpu_sc as plsc

L_SEQ = 200
B = 4096
D = 32
V = 1000000
LANES = 16
NBUF = 8

CB = 2048
WB = 4 * CB
GRID = -(-V // WB)
YROWS = GRID * CB
VPAD = YROWS * 4

_info = plsc.get_sparse_core_info()
NC = _info.num_cores
NS = _info.num_subcores
NW = NC * NS
BW = B // NW


def _repack_body(t_ref, out_ref):
    x = t_ref[...]
    x4 = jnp.concatenate(
        [x[:, q * CB:(q + 1) * CB] for q in range(4)], axis=0)
    out_ref[...] = x4.T


def _remap_row(idx_v, i):
    for j in range(BW // LANES):
        sl = (i, pl.ds(j * LANES, LANES))
        v = idx_v[sl]
        g = v >> 13
        q = (v >> 11) & 3
        off = v & (CB - 1)
        idx_v[sl] = (((g * CB + off) << 2) + q).astype(jnp.int32)


def _accumulate(acc, buf):
    for i in range(BW):
        for j in range(D // LANES):
            sl = (i, pl.ds(j * LANES, LANES))
            plsc.addupdate(acc.at[sl], buf[sl])


def _sc_body(pad_hbm, table_hbm, out_hbm, idx_v, bufs, acc, sems):
    wid = lax.axis_index("s") * NC + lax.axis_index("c")
    base = wid * BW

    pltpu.sync_copy(pad_hbm.at[:, pl.ds(base, BW)], idx_v)

    zeros = jnp.zeros((LANES,), jnp.float32)
    for i in range(BW):
        for j in range(D // LANES):
            acc[i, pl.ds(j * LANES, LANES)] = zeros

    for b in range(NBUF):
        _remap_row(idx_v, b)
        pltpu.async_copy(table_hbm.at[idx_v.at[b]], bufs.at[b], sems.at[b])

    n_groups = L_SEQ // NBUF

    @pl.loop(0, n_groups)
    def _group(g):
        l0 = g * NBUF
        for b in range(NBUF):
            pltpu.make_async_copy(
                table_hbm.at[idx_v.at[0]], bufs.at[b], sems.at[b]).wait()
            _accumulate(acc, bufs.at[b])

            @pl.when(l0 + b + NBUF < L_SEQ)
            def _refill():
                _remap_row(idx_v, l0 + b + NBUF)
                pltpu.async_copy(
                    table_hbm.at[idx_v.at[l0 + b + NBUF]], bufs.at[b],
                    sems.at[b])

    pltpu.sync_copy(acc, out_hbm.at[pl.ds(base, BW)])


def _tc_body(sum_ref, len_ref, out_ref):
    out_ref[...] = sum_ref[...].T * len_ref[...]


@jax.jit
def _product_encoder(product_pad, product_len, emb_table):
    table_t = emb_table.T
    repack = pl.pallas_call(
        _repack_body,
        grid=(GRID,),
        in_specs=[pl.BlockSpec((D, WB), lambda i: (0, i))],
        out_specs=pl.BlockSpec((CB, 4 * D), lambda i: (i, 0)),
        out_shape=jax.ShapeDtypeStruct((YROWS, 4 * D), jnp.float32),
    )
    table_y = repack(table_t).reshape(VPAD, D)

    mesh = plsc.VectorSubcoreMesh(core_axis_name="c", subcore_axis_name="s")
    gather_sum = pl.kernel(
        _sc_body,
        out_type=jax.ShapeDtypeStruct((B, D), jnp.float32),
        mesh=mesh,
        compiler_params=pltpu.CompilerParams(use_tc_tiling_on_sc=False),
        scratch_types=[
            pltpu.VMEM((L_SEQ, BW), jnp.int32),
            pltpu.VMEM((NBUF, BW, D), jnp.float32),
            pltpu.VMEM((BW, D), jnp.float32),
            pltpu.SemaphoreType.DMA((NBUF,)),
        ],
    )
    sums = gather_sum(product_pad, table_y)

    scale_t = pl.pallas_call(
        _tc_body,
        out_shape=jax.ShapeDtypeStruct((D, B), jnp.float32),
    )
    return scale_t(sums, product_len.reshape(1, B))


def kernel(product_pad, product_len, emb_table):
    return _product_encoder(
        product_pad.astype(jnp.int32), product_len, emb_table)


# speedup vs baseline: 1.1812x; 1.1812x over previous
import functools

import jax
import jax.numpy as jnp
from jax import lax
from jax.experimental import pallas as pl
from jax.experimental.pallas import tpu as pltpu
from jax.experimental.pallas import tpu_sc as plsc

L_SEQ = 200
B = 4096
D = 32
V = 1000000
LANES = 16
NBUF = 4

CB = 4096
SH = CB.bit_length() - 1
WB = 4 * CB
GRID = -(-V // WB)
YROWS = GRID * CB
VPAD = YROWS * 4

_info = plsc.get_sparse_core_info()
NC = _info.num_cores
NS = _info.num_subcores
NW = NC * NS
BW = B // NW


def _repack_body(t_ref, out_ref):
    x = t_ref[...]
    x4 = jnp.concatenate(
        [x[:, q * CB:(q + 1) * CB] for q in range(4)], axis=0)
    out_ref[...] = x4.T


def _remap_row(idx_v, i):
    for j in range(BW // LANES):
        sl = (i, pl.ds(j * LANES, LANES))
        v = idx_v[sl]
        g = v >> (SH + 2)
        q = (v >> SH) & 3
        off = v & (CB - 1)
        idx_v[sl] = (((g * CB + off) << 2) + q).astype(jnp.int32)


def _accumulate(acc, buf):
    for i in range(BW):
        for j in range(D // LANES):
            sl = (i, pl.ds(j * LANES, LANES))
            plsc.addupdate(acc.at[sl], buf[sl])


def _sc_body(pad_hbm, table_hbm, out_hbm, idx_v, bufs, acc, sems):
    wid = lax.axis_index("s") * NC + lax.axis_index("c")
    base = wid * BW

    pltpu.sync_copy(pad_hbm.at[:, pl.ds(base, BW)], idx_v)

    zeros = jnp.zeros((LANES,), jnp.float32)
    for i in range(BW):
        for j in range(D // LANES):
            acc[i, pl.ds(j * LANES, LANES)] = zeros

    for b in range(NBUF):
        _remap_row(idx_v, b)
        pltpu.async_copy(table_hbm.at[idx_v.at[b]], bufs.at[b], sems.at[b])

    n_groups = L_SEQ // NBUF

    @pl.loop(0, n_groups)
    def _group(g):
        l0 = g * NBUF
        for b in range(NBUF):
            pltpu.make_async_copy(
                table_hbm.at[idx_v.at[0]], bufs.at[b], sems.at[b]).wait()
            _accumulate(acc, bufs.at[b])

            @pl.when(l0 + b + NBUF < L_SEQ)
            def _refill():
                _remap_row(idx_v, l0 + b + NBUF)
                pltpu.async_copy(
                    table_hbm.at[idx_v.at[l0 + b + NBUF]], bufs.at[b],
                    sems.at[b])

    pltpu.sync_copy(acc, out_hbm.at[pl.ds(base, BW)])


def _tc_body(sum_ref, len_ref, out_ref):
    out_ref[...] = sum_ref[...].T * len_ref[...]


@jax.jit
def _product_encoder(product_pad, product_len, emb_table):
    table_t = emb_table.T
    repack = pl.pallas_call(
        _repack_body,
        grid=(GRID,),
        in_specs=[pl.BlockSpec((D, WB), lambda i: (0, i))],
        out_specs=pl.BlockSpec((CB, 4 * D), lambda i: (i, 0)),
        out_shape=jax.ShapeDtypeStruct((YROWS, 4 * D), jnp.float32),
    )
    table_y = repack(table_t).reshape(VPAD, D)

    mesh = plsc.VectorSubcoreMesh(core_axis_name="c", subcore_axis_name="s")
    gather_sum = pl.kernel(
        _sc_body,
        out_type=jax.ShapeDtypeStruct((B, D), jnp.float32),
        mesh=mesh,
        compiler_params=pltpu.CompilerParams(use_tc_tiling_on_sc=False),
        scratch_types=[
            pltpu.VMEM((L_SEQ, BW), jnp.int32),
            pltpu.VMEM((NBUF, BW, D), jnp.float32),
            pltpu.VMEM((BW, D), jnp.float32),
            pltpu.SemaphoreType.DMA((NBUF,)),
        ],
    )
    sums = gather_sum(product_pad, table_y)

    scale_t = pl.pallas_call(
        _tc_body,
        out_shape=jax.ShapeDtypeStruct((D, B), jnp.float32),
    )
    return scale_t(sums, product_len.reshape(1, B))


def kernel(product_pad, product_len, emb_table):
    return _product_encoder(
        product_pad.astype(jnp.int32), product_len, emb_table)
